# baseline (speedup 1.0000x reference)
import functools

import jax
import jax.numpy as jnp
from jax import lax
from jax.experimental import pallas as pl
from jax.experimental.pallas import tpu as pltpu
from jax.experimental.pallas import tpu_sc as plsc

SBLK = 128
NBUF = 4


@functools.lru_cache(maxsize=None)
def _build_gather(Vp, D, S, T):
    info = plsc.get_sparse_core_info()
    NC, NS = info.num_cores, info.num_subcores
    NW = NC * NS
    assert S == NW * SBLK and D == 64
    assert T % NBUF == 0
    n_groups = T // NBUF

    mesh = plsc.VectorSubcoreMesh(core_axis_name="c", subcore_axis_name="s")

    @functools.partial(
        pl.kernel,
        mesh=mesh,
        out_type=jax.ShapeDtypeStruct((T, D, S), jnp.float32),
        compiler_params=pltpu.CompilerParams(
            use_tc_tiling_on_sc=True, needs_layout_passes=False),
        scratch_types=(
            [pltpu.VMEM((T, SBLK), jnp.int32)]
            + [pltpu.VMEM((SBLK, 2 * D), jnp.float32) for _ in range(NBUF)]
            + [pltpu.VMEM((D, SBLK), jnp.float32) for _ in range(NBUF)]
            + [pltpu.SemaphoreType.DMA for _ in range(2 * NBUF)]
        ),
    )
    def gather_kernel(tpad_hbm, xt_hbm, out_hbm, idx_v, *scratch):
        rows = scratch[:NBUF]
        trans = scratch[NBUF:2 * NBUF]
        gsem = scratch[2 * NBUF:3 * NBUF]
        wsem = scratch[3 * NBUF:]
        wid = lax.axis_index("s") * NC + lax.axis_index("c")
        s0 = wid * SBLK

        pltpu.sync_copy(xt_hbm.at[:, pl.ds(s0, SBLK)], idx_v)

        lane = lax.iota(jnp.int32, 16)

        def start_gather(t, b):
            pltpu.make_async_copy(
                tpad_hbm.at[idx_v.at[t]], rows[b], gsem[b]).start()

        def wait_gather(b):
            pltpu.make_async_copy(
                tpad_hbm.at[idx_v.at[0]], rows[b], gsem[b]).wait()

        def start_write(t, b):
            pltpu.make_async_copy(
                trans[b], out_hbm.at[t, :, pl.ds(s0, SBLK)], wsem[b]).start()

        def wait_write(b):
            pltpu.make_async_copy(
                trans[b], out_hbm.at[0, :, pl.ds(s0, SBLK)], wsem[b]).wait()

        def transpose(b):
            def body(d, carry):
                col = jnp.full((16,), d, dtype=jnp.int32)
                for j in range(SBLK // 16):
                    r = j * 16 + lane
                    v = plsc.load_gather(rows[b], [r, col])
                    trans[b][d, pl.ds(j * 16, 16)] = v
                return carry
            lax.fori_loop(0, D, body, 0)

        for b in range(NBUF):
            start_gather(b, b)

        def group(g, carry):
            t0 = g * NBUF
            for b in range(NBUF):
                wait_gather(b)
                wait_write(b)
                transpose(b)
                start_write(t0 + b, b)
                start_gather(t0 + NBUF + b, b)
            return carry

        for b in range(NBUF):
            wait_gather(b)
            transpose(b)
            start_write(b, b)
            start_gather(NBUF + b, b)

        lax.fori_loop(1, n_groups - 1, group, 0)

        t0 = (n_groups - 1) * NBUF
        for b in range(NBUF):
            wait_gather(b)
            wait_write(b)
            transpose(b)
            start_write(t0 + b, b)
        for b in range(NBUF):
            wait_write(b)

    return gather_kernel


def kernel(x, table):
    V, D = table.shape
    S, T = x.shape
    tpad = jnp.pad(table, ((0, 0), (0, 2 * D - 128 + 64)))
    xt = x.T.astype(jnp.int32)
    fn = _build_gather(tpad.shape[0], D, S, T)
    out = fn(tpad, xt)
    return jnp.transpose(out, (2, 0, 1))


# speedup vs baseline: 1.5533x; 1.5533x over previous
import functools

import jax
import jax.numpy as jnp
from jax import lax
from jax.experimental import pallas as pl
from jax.experimental.pallas import tpu as pltpu
from jax.experimental.pallas import tpu_sc as plsc

SBLK = 128
NBUF = 4


@functools.lru_cache(maxsize=None)
def _build_gather(Vp, D, S, T):
    info = plsc.get_sparse_core_info()
    NC, NS = info.num_cores, info.num_subcores
    NW = NC * NS
    assert S == NW * SBLK and D == 64
    assert T % NBUF == 0
    n_groups = T // NBUF

    mesh = plsc.VectorSubcoreMesh(core_axis_name="c", subcore_axis_name="s")

    @functools.partial(
        pl.kernel,
        mesh=mesh,
        out_type=jax.ShapeDtypeStruct((T, D, S), jnp.float32),
        compiler_params=pltpu.CompilerParams(
            use_tc_tiling_on_sc=True, needs_layout_passes=False),
        scratch_types=(
            [pltpu.VMEM((T, SBLK), jnp.int32)]
            + [pltpu.VMEM((SBLK, 2 * D), jnp.float32) for _ in range(NBUF)]
            + [pltpu.VMEM((D, SBLK), jnp.float32) for _ in range(NBUF)]
            + [pltpu.SemaphoreType.DMA for _ in range(2 * NBUF)]
        ),
    )
    def gather_kernel(tpad_hbm, xt_hbm, out_hbm, idx_v, *scratch):
        rows = scratch[:NBUF]
        trans = scratch[NBUF:2 * NBUF]
        gsem = scratch[2 * NBUF:3 * NBUF]
        wsem = scratch[3 * NBUF:]
        wid = lax.axis_index("s") * NC + lax.axis_index("c")
        s0 = wid * SBLK

        pltpu.sync_copy(xt_hbm.at[:, pl.ds(s0, SBLK)], idx_v)

        lane = lax.iota(jnp.int32, 16)

        def start_gather(t, b):
            pltpu.make_async_copy(
                tpad_hbm.at[idx_v.at[t]], rows[b], gsem[b]).start()

        def wait_gather(b):
            pltpu.make_async_copy(
                tpad_hbm.at[idx_v.at[0]], rows[b], gsem[b]).wait()

        def start_write(t, b):
            pltpu.make_async_copy(
                trans[b], out_hbm.at[t, :, pl.ds(s0, SBLK)], wsem[b]).start()

        def wait_write(b):
            pltpu.make_async_copy(
                trans[b], out_hbm.at[0, :, pl.ds(s0, SBLK)], wsem[b]).wait()

        def transpose(b):
            perm = [(lane + k) & 15 for k in range(16)]
            def body(j, carry):
                rvec = j * 16 + lane
                for d0 in range(0, D, 16):
                    for k in range(16):
                        cvec = perm[k] + d0
                        v = plsc.load_gather(rows[b], [rvec, cvec])
                        plsc.store_scatter(trans[b], [cvec, rvec], v)
                return carry
            lax.fori_loop(0, SBLK // 16, body, 0)

        for b in range(NBUF):
            start_gather(b, b)

        def group(g, carry):
            t0 = g * NBUF
            for b in range(NBUF):
                wait_gather(b)
                wait_write(b)
                transpose(b)
                start_write(t0 + b, b)
                start_gather(t0 + NBUF + b, b)
            return carry

        for b in range(NBUF):
            wait_gather(b)
            transpose(b)
            start_write(b, b)
            start_gather(NBUF + b, b)

        lax.fori_loop(1, n_groups - 1, group, 0)

        t0 = (n_groups - 1) * NBUF
        for b in range(NBUF):
            wait_gather(b)
            wait_write(b)
            transpose(b)
            start_write(t0 + b, b)
        for b in range(NBUF):
            wait_write(b)

    return gather_kernel


def kernel(x, table):
    V, D = table.shape
    S, T = x.shape
    tpad = jnp.pad(table, ((0, 0), (0, 2 * D - 128 + 64)))
    xt = x.T.astype(jnp.int32)
    fn = _build_gather(tpad.shape[0], D, S, T)
    out = fn(tpad, xt)
    return jnp.transpose(out, (2, 0, 1))
